# baseline (speedup 1.0000x reference)
import functools

import jax
import jax.numpy as jnp
from jax import lax
from jax.experimental import pallas as pl
from jax.experimental.pallas import tpu as pltpu
from jax.experimental.pallas import tpu_sc as plsc

BATCH = 4096
ND = 13
NF = 26
NV = 100000
NE = 16
KFM = 8
FN = ND + NF * NE
H1, H2, H3 = 256, 128, 64

NPLANE = NF * NE
ALIGNED = 99968
TAILW = NV - ALIGNED
CW = 4224
NCH = 24
LASTBASE = ALIGNED - CW
MAGIC = 1986
SLOT = 336
NSTEP = 2 * NCH

_sc_mesh = plsc.VectorSubcoreMesh(core_axis_name="c", subcore_axis_name="s")


@functools.partial(
    pl.kernel,
    mesh=_sc_mesh,
    out_type=jax.ShapeDtypeStruct((NPLANE, BATCH), jnp.float32),
    scratch_types=[
        pltpu.VMEM((8, CW), jnp.float32),
        pltpu.VMEM((8, CW), jnp.float32),
        pltpu.VMEM((BATCH,), jnp.float32),
        pltpu.VMEM((BATCH,), jnp.int32),
        pltpu.VMEM((BATCH,), jnp.int32),
        pltpu.VMEM((TAILW * NE,), jnp.float32),
        pltpu.VMEM((NCH, SLOT + 16), jnp.int32),
        pltpu.VMEM((8, BATCH), jnp.float32),
        pltpu.SMEM((NCH,), jnp.int32),
        pltpu.SemaphoreType.DMA,
        pltpu.SemaphoreType.DMA,
    ],
    compiler_params=pltpu.CompilerParams(needs_layout_passes=False),
)
def _sc_gather(tab_hbm, idxT_hbm, tail_hbm, out_hbm,
               s0, s1, idxf_v, idx_v, cid_v, tail_v, bkt_v, row_v, cnt_s,
               sem0, sem1):
    wid = lax.axis_index("s") * 2 + lax.axis_index("c")

    @pl.when(wid < NF)
    def _work():
        f = wid
        pltpu.sync_copy(idxT_hbm.at[f], idxf_v)
        pltpu.sync_copy(tail_hbm.at[pl.ds(f * TAILW * NE, TAILW * NE)], tail_v)

        lanes = lax.iota(jnp.int32, 16)
        zeros = lanes - lanes

        def conv(i, _):
            sl = pl.ds(i * 16, 16)
            v = idxf_v[sl].astype(jnp.int32)
            idx_v[sl] = v
            cid_v[sl] = jnp.minimum(
                ((v >> 7) * MAGIC) >> 16, jnp.int32(NCH - 1))
            return 0
        lax.fori_loop(0, BATCH // 16, conv, 0)

        def chunk_base(c):
            return jnp.where(c == NCH - 1, jnp.int32(LASTBASE), c * CW)

        def fire(k, strip, sem):
            oct_ = k // NCH
            c = k % NCH
            pltpu.async_copy(
                tab_hbm.at[pl.ds(f * NE + oct_ * 8, 8),
                           pl.ds(chunk_base(c), CW)],
                strip, sem)

        def drain(strip, sem):
            pltpu.make_async_copy(
                tab_hbm.at[pl.ds(0, 8), pl.ds(0, CW)], strip, sem).wait()

        fire(0, s0, sem0)

        def bucket_chunk(c, _):
            def scan(i, off):
                sl = pl.ds(i * 16, 16)
                m = cid_v[sl] == c
                plsc.store_compressed(
                    bkt_v.at[c, pl.ds(off, 16)], i * 16 + lanes, mask=m)
                return off + plsc.all_reduce_population_count(m)[0]
            n = lax.fori_loop(0, BATCH // 16, scan, jnp.int32(0))
            cnt_s[c] = n
            return 0
        lax.fori_loop(0, NCH, bucket_chunk, 0)

        def step(k, _):
            oct_ = k // NCH
            c = k % NCH

            @pl.when(k + 1 < NSTEP)
            def _fire_next():
                @pl.when(((k + 1) & 1) == 0)
                def _f0():
                    fire(k + 1, s0, sem0)

                @pl.when(((k + 1) & 1) == 1)
                def _f1():
                    fire(k + 1, s1, sem1)

            n = cnt_s[c]
            base = chunk_base(c)

            def make_extract(strip):
                def extract(j, _):
                    sl = pl.ds(j * 16, 16)
                    b = bkt_v[c, sl] & jnp.int32(BATCH - 1)
                    m = (j * 16 + lanes) < n
                    v = plsc.load_gather(idx_v, [b])
                    loc = jnp.clip(v - base, 0, CW - 1)
                    for es in range(8):
                        sval = plsc.load_gather(strip, [zeros + es, loc])
                        plsc.store_scatter(
                            row_v, [zeros + es, b], sval, mask=m)
                    return 0
                return extract

            def tail_fix(j, _):
                sl = pl.ds(j * 16, 16)
                b = bkt_v[c, sl] & jnp.int32(BATCH - 1)
                v = plsc.load_gather(idx_v, [b])
                m = ((j * 16 + lanes) < n) & (v >= ALIGNED)
                tbase = jnp.clip(v - ALIGNED, 0, TAILW - 1) * NE + oct_ * 8
                for es in range(8):
                    tval = plsc.load_gather(tail_v, [tbase + es])
                    plsc.store_scatter(
                        row_v, [zeros + es, b], tval, mask=m)
                return 0

            nj = (n + 15) >> 4

            @pl.when((k & 1) == 0)
            def _u0():
                drain(s0, sem0)
                lax.fori_loop(0, nj, make_extract(s0), 0)

            @pl.when((k & 1) == 1)
            def _u1():
                drain(s1, sem1)
                lax.fori_loop(0, nj, make_extract(s1), 0)

            @pl.when(c == NCH - 1)
            def _flush():
                lax.fori_loop(0, nj, tail_fix, 0)
                pltpu.sync_copy(
                    row_v, out_hbm.at[pl.ds(f * NE + oct_ * 8, 8)])

            return 0

        lax.fori_loop(0, NSTEP, step, 0)


TB = 512
GRID = BATCH // TB


def _tc_body(d_ref, sT_ref, w0_ref, wd_ref, ws_ref, vd_ref, vs_ref,
             W1d_ref, W1s_ref, b1_ref, W2_ref, b2_ref, W3_ref, b3_ref,
             Wo_ref, bo_ref, o_ref):
    dotT = lambda a, b: lax.dot_general(
        a, b, (((0,), (0,)), ((), ())), preferred_element_type=jnp.float32)
    dot = lambda a, b: jnp.dot(a, b, preferred_element_type=jnp.float32)
    d = d_ref[...]
    sT = sT_ref[...]
    lin = dot(d, wd_ref[...]) + dotT(sT, ws_ref[...]) + w0_ref[0, 0]
    vd = vd_ref[...]
    vs = vs_ref[...]
    xv = dot(d, vd) + dotT(sT, vs)
    x2v2 = dot(d * d, vd * vd) + dotT(sT * sT, vs * vs)
    inter = 0.5 * jnp.sum(xv * xv - x2v2, axis=-1, keepdims=True)
    fm = jax.nn.sigmoid(lin + inter)
    h = jnp.maximum(dot(d, W1d_ref[...]) + dotT(sT, W1s_ref[...])
                    + b1_ref[...], 0.0)
    h = jnp.maximum(dot(h, W2_ref[...]) + b2_ref[...], 0.0)
    h = jnp.maximum(dot(h, W3_ref[...]) + b3_ref[...], 0.0)
    deep = dot(h, Wo_ref[...]) + bo_ref[0, 0]
    o_ref[...] = jax.nn.sigmoid(0.5 * (fm + deep))


def _full(shape):
    return pl.BlockSpec(shape, lambda i: (0, 0))


_tc_dense = pl.pallas_call(
    _tc_body,
    grid=(GRID,),
    in_specs=[
        pl.BlockSpec((TB, ND), lambda i: (i, 0)),
        pl.BlockSpec((NPLANE, TB), lambda i: (0, i)),
        _full((1, 1)), _full((ND, 1)), _full((NPLANE, 1)),
        _full((ND, KFM)), _full((NPLANE, KFM)),
        _full((ND, H1)), _full((NPLANE, H1)), _full((1, H1)),
        _full((H1, H2)), _full((1, H2)),
        _full((H2, H3)), _full((1, H3)),
        _full((H3, 1)), _full((1, 1)),
    ],
    out_specs=pl.BlockSpec((TB, 1), lambda i: (i, 0)),
    out_shape=jax.ShapeDtypeStruct((BATCH, 1), jnp.float32),
)


def kernel(inputs, emb_tables, w0, w, v, W1, b1, W2, b2, W3, b3, Wo, bo):
    d = inputs[:, :ND]
    idxT = inputs[:, ND:].T
    tabT = emb_tables.transpose(0, 2, 1).reshape(NPLANE, NV)
    tail = emb_tables[:, ALIGNED:, :].reshape(NF * TAILW * NE)
    sT = _sc_gather(tabT, idxT, tail)
    return _tc_dense(
        d, sT, w0.reshape(1, 1), w[:ND], w[ND:], v[:ND], v[ND:],
        W1[:ND], W1[ND:], b1.reshape(1, H1), W2, b2.reshape(1, H2),
        W3, b3.reshape(1, H3), Wo, bo.reshape(1, 1))


# speedup vs baseline: 1.2196x; 1.2196x over previous
import functools

import jax
import jax.numpy as jnp
from jax import lax
from jax.experimental import pallas as pl
from jax.experimental.pallas import tpu as pltpu
from jax.experimental.pallas import tpu_sc as plsc

BATCH = 4096
ND = 13
NF = 26
NV = 100000
NE = 16
KFM = 8
FN = ND + NF * NE
H1, H2, H3 = 256, 128, 64

NPLANE = NF * NE
ALIGNED = 99968
TAILW = NV - ALIGNED
CW = 4224
NCH = 24
LASTBASE = ALIGNED - CW
MAGIC = 1986
SLOT = 336
NSTEP = 2 * NCH

_sc_mesh = plsc.VectorSubcoreMesh(core_axis_name="c", subcore_axis_name="s")


@functools.partial(
    pl.kernel,
    mesh=_sc_mesh,
    out_type=jax.ShapeDtypeStruct((NPLANE, BATCH), jnp.float32),
    scratch_types=[
        pltpu.VMEM((8, CW), jnp.float32),
        pltpu.VMEM((8, CW), jnp.float32),
        pltpu.VMEM((BATCH,), jnp.float32),
        pltpu.VMEM((BATCH,), jnp.int32),
        pltpu.VMEM((BATCH,), jnp.int32),
        pltpu.VMEM((TAILW * NE,), jnp.float32),
        pltpu.VMEM((NCH, SLOT + 16), jnp.int32),
        pltpu.VMEM((4, 1296), jnp.int32),
        pltpu.VMEM((8, BATCH), jnp.float32),
        pltpu.SMEM((4 + NCH,), jnp.int32),
        pltpu.SemaphoreType.DMA,
        pltpu.SemaphoreType.DMA,
    ],
    compiler_params=pltpu.CompilerParams(needs_layout_passes=False),
)
def _sc_gather(tab_hbm, idxT_hbm, tail_hbm, out_hbm,
               s0, s1, idxf_v, idx_v, cid_v, tail_v, bkt_v, sup_v, row_v, cnt_s,
               sem0, sem1):
    wid = lax.axis_index("s") * 2 + lax.axis_index("c")

    @pl.when(wid < NF)
    def _work():
        f = wid
        pltpu.sync_copy(idxT_hbm.at[f], idxf_v)
        pltpu.sync_copy(tail_hbm.at[pl.ds(f * TAILW * NE, TAILW * NE)], tail_v)

        lanes = lax.iota(jnp.int32, 16)
        zeros = lanes - lanes

        def conv(i, _):
            sl = pl.ds(i * 16, 16)
            v = idxf_v[sl].astype(jnp.int32)
            idx_v[sl] = v
            cid_v[sl] = jnp.minimum(
                ((v >> 7) * MAGIC) >> 16, jnp.int32(NCH - 1))
            return 0
        lax.fori_loop(0, BATCH // 16, conv, 0)

        def chunk_base(c):
            return jnp.where(c == NCH - 1, jnp.int32(LASTBASE), c * CW)

        def fire(k, strip, sem):
            oct_ = k // NCH
            c = k % NCH
            pltpu.async_copy(
                tab_hbm.at[pl.ds(f * NE + oct_ * 8, 8),
                           pl.ds(chunk_base(c), CW)],
                strip, sem)

        def drain(strip, sem):
            pltpu.make_async_copy(
                tab_hbm.at[pl.ds(0, 8), pl.ds(0, CW)], strip, sem).wait()

        fire(0, s0, sem0)

        def super_pass(sb, _):
            def scan(i, off):
                sl = pl.ds(i * 16, 16)
                cid = cid_v[sl]
                m = ((cid * 43) >> 8) == sb
                packed = (i * 16 + lanes) | (cid << 12)
                plsc.store_compressed(
                    sup_v.at[sb, pl.ds(off, 16)], packed, mask=m)
                return off + plsc.all_reduce_population_count(m)[0]
            n = lax.fori_loop(0, BATCH // 16, scan, jnp.int32(0))
            cnt_s[sb] = n
            return 0
        lax.fori_loop(0, 4, super_pass, 0)

        def sub_pass(c, _):
            sb = (c * 43) >> 8
            ns = cnt_s[sb]

            def scan(i, off):
                sl = pl.ds(i * 16, 16)
                e = sup_v[sb, sl]
                m = ((i * 16 + lanes) < ns) & ((e >> 12) == c)
                plsc.store_compressed(
                    bkt_v.at[c, pl.ds(off, 16)], e & jnp.int32(4095), mask=m)
                return off + plsc.all_reduce_population_count(m)[0]
            n = lax.fori_loop(0, (ns + 15) >> 4, scan, jnp.int32(0))
            cnt_s[4 + c] = n
            return 0
        lax.fori_loop(0, NCH, sub_pass, 0)

        def step(k, _):
            oct_ = k // NCH
            c = k % NCH

            @pl.when(k + 1 < NSTEP)
            def _fire_next():
                @pl.when(((k + 1) & 1) == 0)
                def _f0():
                    fire(k + 1, s0, sem0)

                @pl.when(((k + 1) & 1) == 1)
                def _f1():
                    fire(k + 1, s1, sem1)

            n = cnt_s[4 + c]
            base = chunk_base(c)

            def make_extract(strip):
                def extract(j, _):
                    sl = pl.ds(j * 16, 16)
                    b = bkt_v[c, sl] & jnp.int32(BATCH - 1)
                    m = (j * 16 + lanes) < n
                    v = plsc.load_gather(idx_v, [b])
                    loc = jnp.clip(v - base, 0, CW - 1)
                    for es in range(8):
                        sval = plsc.load_gather(strip, [zeros + es, loc])
                        plsc.store_scatter(
                            row_v, [zeros + es, b], sval, mask=m)
                    return 0
                return extract

            def tail_fix(j, _):
                sl = pl.ds(j * 16, 16)
                b = bkt_v[c, sl] & jnp.int32(BATCH - 1)
                v = plsc.load_gather(idx_v, [b])
                m = ((j * 16 + lanes) < n) & (v >= ALIGNED)
                tbase = jnp.clip(v - ALIGNED, 0, TAILW - 1) * NE + oct_ * 8
                for es in range(8):
                    tval = plsc.load_gather(tail_v, [tbase + es])
                    plsc.store_scatter(
                        row_v, [zeros + es, b], tval, mask=m)
                return 0

            nj = (n + 15) >> 4

            @pl.when((k & 1) == 0)
            def _u0():
                drain(s0, sem0)
                lax.fori_loop(0, nj, make_extract(s0), 0)

            @pl.when((k & 1) == 1)
            def _u1():
                drain(s1, sem1)
                lax.fori_loop(0, nj, make_extract(s1), 0)

            @pl.when(c == NCH - 1)
            def _flush():
                lax.fori_loop(0, nj, tail_fix, 0)
                pltpu.sync_copy(
                    row_v, out_hbm.at[pl.ds(f * NE + oct_ * 8, 8)])

            return 0

        lax.fori_loop(0, NSTEP, step, 0)


TB = 512
GRID = BATCH // TB


def _tc_body(d_ref, sT_ref, w0_ref, wd_ref, ws_ref, vd_ref, vs_ref,
             W1d_ref, W1s_ref, b1_ref, W2_ref, b2_ref, W3_ref, b3_ref,
             Wo_ref, bo_ref, o_ref):
    dotT = lambda a, b: lax.dot_general(
        a, b, (((0,), (0,)), ((), ())), preferred_element_type=jnp.float32)
    dot = lambda a, b: jnp.dot(a, b, preferred_element_type=jnp.float32)
    d = d_ref[...]
    sT = sT_ref[...]
    lin = dot(d, wd_ref[...]) + dotT(sT, ws_ref[...]) + w0_ref[0, 0]
    vd = vd_ref[...]
    vs = vs_ref[...]
    xv = dot(d, vd) + dotT(sT, vs)
    x2v2 = dot(d * d, vd * vd) + dotT(sT * sT, vs * vs)
    inter = 0.5 * jnp.sum(xv * xv - x2v2, axis=-1, keepdims=True)
    fm = jax.nn.sigmoid(lin + inter)
    h = jnp.maximum(dot(d, W1d_ref[...]) + dotT(sT, W1s_ref[...])
                    + b1_ref[...], 0.0)
    h = jnp.maximum(dot(h, W2_ref[...]) + b2_ref[...], 0.0)
    h = jnp.maximum(dot(h, W3_ref[...]) + b3_ref[...], 0.0)
    deep = dot(h, Wo_ref[...]) + bo_ref[0, 0]
    o_ref[...] = jax.nn.sigmoid(0.5 * (fm + deep))


def _full(shape):
    return pl.BlockSpec(shape, lambda i: (0, 0))


_tc_dense = pl.pallas_call(
    _tc_body,
    grid=(GRID,),
    in_specs=[
        pl.BlockSpec((TB, ND), lambda i: (i, 0)),
        pl.BlockSpec((NPLANE, TB), lambda i: (0, i)),
        _full((1, 1)), _full((ND, 1)), _full((NPLANE, 1)),
        _full((ND, KFM)), _full((NPLANE, KFM)),
        _full((ND, H1)), _full((NPLANE, H1)), _full((1, H1)),
        _full((H1, H2)), _full((1, H2)),
        _full((H2, H3)), _full((1, H3)),
        _full((H3, 1)), _full((1, 1)),
    ],
    out_specs=pl.BlockSpec((TB, 1), lambda i: (i, 0)),
    out_shape=jax.ShapeDtypeStruct((BATCH, 1), jnp.float32),
)


def kernel(inputs, emb_tables, w0, w, v, W1, b1, W2, b2, W3, b3, Wo, bo):
    d = inputs[:, :ND]
    idxT = inputs[:, ND:].T
    tabT = emb_tables.transpose(0, 2, 1).reshape(NPLANE, NV)
    tail = emb_tables[:, ALIGNED:, :].reshape(NF * TAILW * NE)
    sT = _sc_gather(tabT, idxT, tail)
    return _tc_dense(
        d, sT, w0.reshape(1, 1), w[:ND], w[ND:], v[:ND], v[ND:],
        W1[:ND], W1[ND:], b1.reshape(1, H1), W2, b2.reshape(1, H2),
        W3, b3.reshape(1, H3), Wo, bo.reshape(1, 1))
